# baseline (speedup 1.0000x reference)
import functools

import jax
import jax.numpy as jnp
from jax import lax
from jax.experimental import pallas as pl
from jax.experimental.pallas import tpu as pltpu
from jax.experimental.pallas import tpu_sc as plsc

V = 100000
H = 128
P = 512
T = 2
EPS = 1e-12

NC = 2
NS = 16
NW = NC * NS

C = 128
NB = 6
PF = 3

S = 1
LN_TILE = 12800


def _body(ids_r, cids_r, wtab_r, comb_r, out_r, idx_v, cidx_v, comb_sh,
          rows0_v, rows1_v, rows2_v, rows3_v, rows4_v, rows5_v, *sems,
          chunks_per_w):
  wid = lax.axis_index("s") * NC + lax.axis_index("c")
  row0 = wid * chunks_per_w

  pltpu.sync_copy(ids_r.at[wid], idx_v)
  pltpu.sync_copy(cids_r.at[wid], cidx_v)

  @pl.when(lax.axis_index("s") == 0)
  def _():
    pltpu.sync_copy(comb_r, comb_sh)

  plsc.subcore_barrier()

  rows = (rows0_v, rows1_v, rows2_v, rows3_v, rows4_v, rows5_v)
  semc = sems[0:NB]
  semw = sems[NB:2 * NB]
  semo = sems[2 * NB:3 * NB]

  def start_comb(ci, p):
    pltpu.async_copy(comb_sh.at[cidx_v.at[ci]], rows[p], semc[p])

  def wait_comb(p):
    pltpu.make_async_copy(comb_sh.at[pl.ds(0, C)], rows[p], semc[p]).wait()

  def start_word(ci, p):
    pltpu.async_copy(wtab_r.at[idx_v.at[ci]], rows[p], semw[p], add=True)

  def wait_word(p):
    pltpu.make_async_copy(wtab_r.at[pl.ds(0, C)], rows[p], semw[p]).wait()

  def start_out(ci, p):
    pltpu.async_copy(rows[p], out_r.at[pl.ds((row0 + ci) * C, C)], semo[p])

  def wait_out(p):
    pltpu.make_async_copy(rows[p], out_r.at[pl.ds(0, C)], semo[p]).wait()

  for j in range(PF):
    start_comb(j, j)

  main = chunks_per_w - chunks_per_w % NB

  @pl.loop(0, main, step=NB)
  def ring(ci0):
    for k in range(NB):
      ci = ci0 + k
      p = k
      pv = (k - 2) % NB
      q = (k + PF) % NB

      wait_comb(p)
      start_word(ci, p)

      @pl.when(ci >= 2)
      def _():
        wait_word(pv)
        start_out(ci - 2, pv)

      @pl.when(ci >= NB - PF)
      def _():
        wait_out(q)

      @pl.when(ci + PF < chunks_per_w)
      def _():
        start_comb(ci + PF, q)

  for ci in range(main, chunks_per_w):
    p = ci % NB
    pv = (ci - 2) % NB
    q = (ci + PF) % NB
    wait_comb(p)
    start_word(ci, p)
    wait_word(pv)
    start_out(ci - 2, pv)
    wait_out(q)

  for ci in (chunks_per_w - 2, chunks_per_w - 1):
    wait_word(ci % NB)
    start_out(ci, ci % NB)

  for j in range(chunks_per_w - (NB - PF), chunks_per_w):
    wait_out(j % NB)


def _ln_first(x_ref, w_ref, b_ref, o_ref):
  x = x_ref[...]
  mu = jnp.mean(x, axis=-1, keepdims=True)
  xc = x - mu
  var = jnp.mean(xc * xc, axis=-1, keepdims=True)
  o_ref[...] = xc * lax.rsqrt(var + EPS) * w_ref[...] + b_ref[...]


def _ln_next(prev_ref, x_ref, w_ref, b_ref, o_ref):
  del prev_ref
  _ln_first(x_ref, w_ref, b_ref, o_ref)


def _ln_call(prev_out, summed_s, ln_w2, ln_b2, s, n):
  n_slab = summed_s.shape[0]
  tiles = n_slab // LN_TILE
  xwb_specs = [
      pl.BlockSpec((LN_TILE, H), lambda i: (i, 0)),
      pl.BlockSpec((1, H), lambda i: (0, 0)),
      pl.BlockSpec((1, H), lambda i: (0, 0)),
  ]
  out_spec = pl.BlockSpec((LN_TILE, H), lambda i, s=s, t=tiles: (s * t + i, 0))
  out_shape = jax.ShapeDtypeStruct((n, H), jnp.float32)
  if prev_out is None:
    return pl.pallas_call(
        _ln_first,
        grid=(tiles,),
        in_specs=xwb_specs,
        out_specs=out_spec,
        out_shape=out_shape,
    )(summed_s, ln_w2, ln_b2)
  return pl.pallas_call(
      _ln_next,
      grid=(tiles,),
      in_specs=[pl.BlockSpec(memory_space=pl.ANY)] + xwb_specs,
      out_specs=out_spec,
      out_shape=out_shape,
      input_output_aliases={0: 0},
  )(prev_out, summed_s, ln_w2, ln_b2)


@jax.jit
def _run(ids3, cids3, word_table, comb, ln_w2, ln_b2):
  chunks_per_w = ids3.shape[2]
  n_slab = NW * chunks_per_w * C
  n = S * n_slab
  mesh = plsc.VectorSubcoreMesh(core_axis_name="c", subcore_axis_name="s")
  kern = pl.kernel(
      functools.partial(_body, chunks_per_w=chunks_per_w),
      out_type=jax.ShapeDtypeStruct((n_slab, H), jnp.float32),
      mesh=mesh,
      compiler_params=pltpu.CompilerParams(needs_layout_passes=False),
      scratch_types=[
          pltpu.VMEM((chunks_per_w, C), jnp.int32),
          pltpu.VMEM((chunks_per_w, C), jnp.int32),
          pltpu.VMEM_SHARED((T * P, H), jnp.float32),
      ] + [pltpu.VMEM((C, H), jnp.float32)] * NB
        + [pltpu.SemaphoreType.DMA] * (3 * NB),
  )
  out = None
  for s in range(S):
    summed_s = kern(ids3[s], cids3[s], word_table, comb)
    out = _ln_call(out, summed_s, ln_w2, ln_b2, s, n)
  return out


def kernel(input_ids, position_ids, token_type_ids, word_table, pos_table,
           type_table, ln_weight, ln_bias):
  b, l = input_ids.shape
  n = b * l
  cpw = n // (S * NW * C)
  comb = (type_table[:, None, :] + pos_table[None, :, :]).reshape(T * P, H)
  cids = token_type_ids * P + position_ids
  ids3 = input_ids.reshape(S, NW, cpw, C)
  cids3 = cids.reshape(S, NW, cpw, C)
  out = _run(ids3, cids3, word_table, comb, ln_weight.reshape(1, H),
             ln_bias.reshape(1, H))
  return out.reshape(b, l, H)
